# baseline (speedup 1.0000x reference)
import functools

import jax
import jax.numpy as jnp
from jax import lax
from jax.experimental import pallas as pl
from jax.experimental.pallas import tpu as pltpu
from jax.experimental.pallas import tpu_sc as plsc

B = 1024
S = 200
V = 100000
L = 16
NCHUNK = 13
NW = 32
RPW = B // NW
BLK = RPW * S

PIECE_V = 256
NP = (V + PIECE_V - 1) // PIECE_V
PTAB = 416
SCAP = 128
SCAP_SHIFT = 7
FSTR = 10624
NFULL = 97
VLAST = V - (NP - 1) * PIECE_V


def _build_phase_a():
    info = plsc.get_sparse_core_info()
    nc = info.num_cores
    mesh = plsc.VectorSubcoreMesh(core_axis_name="c", subcore_axis_name="s")

    @functools.partial(
        pl.kernel,
        mesh=mesh,
        compiler_params=pltpu.CompilerParams(needs_layout_passes=False),
        out_type=[
            jax.ShapeDtypeStruct((NW * FSTR,), jnp.int32),
            jax.ShapeDtypeStruct((NW * FSTR,), jnp.float32),
            jax.ShapeDtypeStruct((NW * PTAB,), jnp.int32),
        ],
        scratch_types=[
            pltpu.VMEM((S, 128), jnp.int32),
            pltpu.VMEM((S, 128), jnp.float32),
            pltpu.VMEM((PTAB,), jnp.int32),
            pltpu.VMEM((PTAB,), jnp.int32),
            pltpu.VMEM((FSTR,), jnp.int32),
            pltpu.VMEM((FSTR,), jnp.float32),
        ],
    )
    def ka(idx_hbm, w_hbm, a_hbm, wf_hbm, p_hbm, idx_v, w_v, cnt, off, af, wf):
        w = lax.axis_index("s") * nc + lax.axis_index("c")
        lane = lax.broadcasted_iota(jnp.int32, (L,), 0)
        zi = jnp.zeros((L,), jnp.int32)
        zf = jnp.zeros((L,), jnp.float32)
        ones = jnp.ones((L,), jnp.int32)

        d0, _ = plsc.scan_count(lane)
        dbase = jnp.sum(jnp.where(lane == 0, d0, 0))

        bt = w >> 2
        pltpu.sync_copy(idx_hbm.at[:, pl.ds(bt * 128, 128)], idx_v)
        pltpu.sync_copy(w_hbm.at[:, pl.ds(bt * 128, 128)], w_v)

        def zero_loop(ref, n, val):
            def zb(i, c):
                ref[pl.ds(i * L, L)] = val
                return c
            lax.fori_loop(0, n // L, zb, 0)

        zero_loop(cnt, PTAB, zi)
        zero_loop(af, FSTR, zi)
        zero_loop(wf, FSTR, zf)

        col0 = (w & 3) * RPW

        def hist_s(s, c):
            for h in range(2):
                iv = idx_v[s, pl.ds(col0 + h * L, L)]
                pv = iv >> 8
                plsc.addupdate_scatter(cnt, [pv], ones)
            return c
        lax.fori_loop(0, S, hist_s, 0)

        def pfx(k, carry):
            v = cnt[pl.ds(k * L, L)]
            va = (v + 7) & (-8)
            s = plsc.cumsum(va)
            off[pl.ds(k * L, L)] = s - va + carry
            cnt[pl.ds(k * L, L)] = s - va + carry
            return carry + jnp.sum(va)
        lax.fori_loop(0, PTAB // L, pfx, jnp.int32(0))

        def place_s(s, c):
            for h in range(2):
                bl = col0 + h * L + lane
                iv = idx_v[s, pl.ds(col0 + h * L, L)]
                wv = w_v[s, pl.ds(col0 + h * L, L)]
                pv = iv >> 8
                paddr = (iv << 7) | bl
                pos = plsc.load_gather(cnt, [pv])
                dup, lastm = plsc.scan_count(pv)
                slot = pos + dup - dbase
                plsc.store_scatter(af, [slot], paddr)
                plsc.store_scatter(wf, [slot], wv)
                plsc.store_scatter(cnt, [pv], slot + 1, mask=lastm)
            return c
        lax.fori_loop(0, S, place_s, 0)

        pltpu.sync_copy(af, a_hbm.at[pl.ds(w * FSTR, FSTR)])
        pltpu.sync_copy(wf, wf_hbm.at[pl.ds(w * FSTR, FSTR)])
        pltpu.sync_copy(off, p_hbm.at[pl.ds(w * PTAB, PTAB)])

    return ka


def _build_phase_b():
    info = plsc.get_sparse_core_info()
    nc = info.num_cores
    mesh = plsc.VectorSubcoreMesh(core_axis_name="c", subcore_axis_name="s")

    @functools.partial(
        pl.kernel,
        mesh=mesh,
        compiler_params=pltpu.CompilerParams(needs_layout_passes=False),
        out_type=jax.ShapeDtypeStruct((V, B), jnp.float32),
        scratch_types=[
            pltpu.VMEM((PIECE_V, 128), jnp.float32),
            pltpu.VMEM((PIECE_V, 128), jnp.float32),
            pltpu.VMEM((2 * 4 * SCAP,), jnp.int32),
            pltpu.VMEM((2 * 4 * SCAP,), jnp.float32),
            pltpu.VMEM((4 * PTAB,), jnp.int32),
            pltpu.SemaphoreType.DMA,
            pltpu.SemaphoreType.DMA,
            pltpu.SemaphoreType.DMA,
            pltpu.SemaphoreType.DMA,
        ],
    )
    def kb(a_hbm, wf_hbm, p_hbm, out_hbm, acc0, acc1, ast, wst, ptab,
           sg0, sg1, so0, so1):
        u = lax.axis_index("s") * nc + lax.axis_index("c")
        j = u & 7
        i = u >> 3
        lane = lax.broadcasted_iota(jnp.int32, (L,), 0)
        zf = jnp.zeros((L,), jnp.float32)

        for s in range(4):
            pltpu.sync_copy(p_hbm.at[pl.ds((4 * j + s) * PTAB, PTAB)],
                            ptab.at[pl.ds(s * PTAB, PTAB)])

        accs = (acc0, acc1)
        sgs = (sg0, sg1)
        sos = (so0, so1)

        def clear_full(acc):
            def zb(k, c):
                r = k >> 3
                col = (k & 7) * L
                acc[r, pl.ds(col, L)] = zf
                return c
            lax.fori_loop(0, (PIECE_V * 128) // L, zb, 0)

        clear_full(acc0)
        clear_full(acc1)

        def ext(s, p):
            ch = ptab[pl.ds(s * PTAB + (p & (-8)), L)]
            k = p & 7
            st = jnp.sum(jnp.where(lane == k, ch, 0))
            en = jnp.sum(jnp.where(lane == k + 1, ch, 0))
            return st, en - st

        def seg_params(s, p):
            st, ln = ext(s, p)
            return pl.multiple_of(st, 8), ln

        def gather_fire(q, p):
            sg = sgs[q]
            for s in range(4):
                st, ln = seg_params(s, p)
                fb = (4 * j + s) * FSTR

                @pl.when(ln > 0)
                def _():
                    pltpu.async_copy(
                        a_hbm.at[pl.ds(fb + st, SCAP)],
                        ast.at[pl.ds((q * 4 + s) * SCAP, SCAP)], sg)
                    pltpu.async_copy(
                        wf_hbm.at[pl.ds(fb + st, SCAP)],
                        wst.at[pl.ds((q * 4 + s) * SCAP, SCAP)], sg)

        def gather_drain(q, p):
            sg = sgs[q]
            for s in range(4):
                st, ln = seg_params(s, p)
                fb = (4 * j + s) * FSTR

                @pl.when(ln > 0)
                def _():
                    pltpu.make_async_copy(
                        a_hbm.at[pl.ds(fb + st, SCAP)],
                        ast.at[pl.ds((q * 4 + s) * SCAP, SCAP)], sg).wait()
                    pltpu.make_async_copy(
                        wf_hbm.at[pl.ds(fb + st, SCAP)],
                        wst.at[pl.ds((q * 4 + s) * SCAP, SCAP)], sg).wait()

        def scan_slot(q, s, p, nch, coff, add):
            acc = accs[q]
            base = (q * 4 + s) * SCAP + coff

            def cb(c, cc):
                av = ast[pl.ds(base + c * L, L)]
                m = (av >> 15) == p
                lv = (av >> 7) & (PIECE_V - 1)
                lb = av & 127
                if add:
                    wv = wst[pl.ds(base + c * L, L)]
                    plsc.addupdate_scatter(acc, [lv, lb], wv, mask=m)
                else:
                    plsc.store_scatter(acc, [lv, lb], zf, mask=m)
                return cc
            lax.fori_loop(0, nch, cb, 0)

        def scatter_piece(q, p):
            for s in range(4):
                st, ln = seg_params(s, p)
                nch0 = jnp.minimum(SCAP // L, (ln + L - 1) >> 4)
                scan_slot(q, s, p, nch0, 0, True)
                nb = (ln + SCAP - 1) >> SCAP_SHIFT
                fb = (4 * j + s) * FSTR

                def batch(k, c):
                    pltpu.sync_copy(
                        a_hbm.at[pl.ds(fb + st + k * SCAP, SCAP)],
                        ast.at[pl.ds((q * 4 + s) * SCAP, SCAP)])
                    pltpu.sync_copy(
                        wf_hbm.at[pl.ds(fb + st + k * SCAP, SCAP)],
                        wst.at[pl.ds((q * 4 + s) * SCAP, SCAP)])
                    rem = ln - k * SCAP
                    nchk = jnp.minimum(SCAP // L, (rem + L - 1) >> 4)
                    scan_slot(q, s, p, nchk, 0, True)
                    return c
                lax.fori_loop(1, nb, batch, 0)

        def rezero(q, p_prev):
            ovf = jnp.int32(0)
            lens = []
            for s in range(4):
                st, ln = seg_params(s, p_prev)
                lens.append(ln)
                ovf = ovf | (ln > SCAP).astype(jnp.int32)

            @pl.when(ovf == 1)
            def _():
                clear_full(accs[q])

            @pl.when(ovf == 0)
            def _():
                for s in range(4):
                    nch0 = jnp.minimum(SCAP // L, (lens[s] + L - 1) >> 4)
                    scan_slot(q, s, p_prev, nch0, 0, False)

        def out_fire(q, p, nrows):
            v0 = p * PIECE_V
            pltpu.async_copy(
                accs[q].at[pl.ds(0, nrows), :],
                out_hbm.at[pl.ds(v0, nrows), pl.ds(j * 128, 128)], sos[q])

        def out_drain(q, nrows):
            pltpu.make_async_copy(
                accs[q].at[pl.ds(0, nrows), :],
                out_hbm.at[pl.ds(0, nrows), pl.ds(j * 128, 128)],
                sos[q]).wait()

        def piece_of(t):
            return i + 4 * t

        def step(q, t, first):
            p = piece_of(t)
            if not first:
                out_drain(q, PIECE_V)
                rezero(q, p - 8)
            gather_fire(q, p)
            gather_drain(q, p)
            scatter_piece(q, p)
            out_fire(q, p, PIECE_V)

        step(0, jnp.int32(0), True)
        step(1, jnp.int32(1), True)

        def pair(t2, c):
            t = 2 + t2 * 2
            step(0, t, False)
            step(1, t + 1, False)
            return c
        lax.fori_loop(0, (NFULL - 3) // 2, pair, 0)
        step(0, jnp.int32(NFULL - 1), False)

        t97 = jnp.int32(NFULL)

        @pl.when(i <= 1)
        def _():
            step(1, t97, False)

        @pl.when(i == 2)
        def _():
            p = piece_of(t97)
            out_drain(1, PIECE_V)
            rezero(1, p - 8)
            gather_fire(1, p)
            gather_drain(1, p)
            scatter_piece(1, p)
            out_fire(1, p, VLAST)

        @pl.when(i <= 1)
        def _():
            out_drain(0, PIECE_V)
            out_drain(1, PIECE_V)

        @pl.when(i == 2)
        def _():
            out_drain(0, PIECE_V)
            out_drain(1, VLAST)

        @pl.when(i == 3)
        def _():
            out_drain(0, PIECE_V)
            out_drain(1, PIECE_V)

    return kb


_phase_a = _build_phase_a()
_phase_b = _build_phase_b()


@jax.jit
def kernel(inputs, count_weights):
    a_file, w_file, p_file = _phase_a(inputs.T, count_weights.T)
    out_t = _phase_b(a_file, w_file, p_file)
    return out_t.T


# speedup vs baseline: 1.0048x; 1.0048x over previous
import functools

import jax
import jax.numpy as jnp
from jax import lax
from jax.experimental import pallas as pl
from jax.experimental.pallas import tpu as pltpu
from jax.experimental.pallas import tpu_sc as plsc

B = 1024
S = 200
V = 100000
L = 16
NCHUNK = 13
NW = 32
RPW = B // NW
BLK = RPW * S

PIECE_V = 256
NP = (V + PIECE_V - 1) // PIECE_V
PTAB = 416
SCAP = 128
SCAP_SHIFT = 7
FSTR = 10624
NFULL = 97
VLAST = V - (NP - 1) * PIECE_V


def _build_phase_a():
    info = plsc.get_sparse_core_info()
    nc = info.num_cores
    mesh = plsc.VectorSubcoreMesh(core_axis_name="c", subcore_axis_name="s")

    @functools.partial(
        pl.kernel,
        mesh=mesh,
        compiler_params=pltpu.CompilerParams(needs_layout_passes=False),
        out_type=[
            jax.ShapeDtypeStruct((NW * FSTR,), jnp.int32),
            jax.ShapeDtypeStruct((NW * FSTR,), jnp.float32),
            jax.ShapeDtypeStruct((NW * PTAB,), jnp.int32),
        ],
        scratch_types=[
            pltpu.VMEM((BLK + 8,), jnp.int32),
            pltpu.VMEM((BLK + 8,), jnp.float32),
            pltpu.VMEM((PTAB,), jnp.int32),
            pltpu.VMEM((PTAB,), jnp.int32),
            pltpu.VMEM((FSTR,), jnp.int32),
            pltpu.VMEM((FSTR,), jnp.float32),
        ],
    )
    def ka(idx_hbm, w_hbm, a_hbm, wf_hbm, p_hbm, idx_v, w_v, cnt, off, af, wf):
        w = lax.axis_index("s") * nc + lax.axis_index("c")
        lane = lax.broadcasted_iota(jnp.int32, (L,), 0)
        mask8 = lane < (S - (NCHUNK - 1) * L)
        zi = jnp.zeros((L,), jnp.int32)
        zf = jnp.zeros((L,), jnp.float32)
        ones = jnp.ones((L,), jnp.int32)

        d0, _ = plsc.scan_count(lane)
        dbase = jnp.sum(jnp.where(lane == 0, d0, 0))

        idx_v[pl.ds(BLK - 8, L)] = zi
        pltpu.sync_copy(idx_hbm.at[pl.ds(w * BLK, BLK)], idx_v.at[pl.ds(0, BLK)])
        pltpu.sync_copy(w_hbm.at[pl.ds(w * BLK, BLK)], w_v.at[pl.ds(0, BLK)])

        def zero_loop(ref, n, val):
            def zb(i, c):
                ref[pl.ds(i * L, L)] = val
                return c
            lax.fori_loop(0, n // L, zb, 0)

        zero_loop(cnt, PTAB, zi)
        zero_loop(af, FSTR, zi)
        zero_loop(wf, FSTR, zf)

        def hist_row(r, c):
            o = r * S
            for ci in range(NCHUNK):
                iv = idx_v[pl.ds(o + ci * L, L)]
                pv = iv >> 8
                if ci == NCHUNK - 1:
                    plsc.addupdate_scatter(cnt, [pv], ones, mask=mask8)
                else:
                    plsc.addupdate_scatter(cnt, [pv], ones)
            return c
        lax.fori_loop(0, RPW, hist_row, 0)

        def pfx(k, carry):
            v = cnt[pl.ds(k * L, L)]
            va = (v + 7) & (-8)
            s = plsc.cumsum(va)
            off[pl.ds(k * L, L)] = s - va + carry
            cnt[pl.ds(k * L, L)] = s - va + carry
            return carry + jnp.sum(va)
        lax.fori_loop(0, PTAB // L, pfx, jnp.int32(0))

        bl0 = (w & 3) * RPW

        def place_row(r, c):
            o = r * S
            bl = bl0 + r
            for ci in range(NCHUNK):
                m = mask8 if ci == NCHUNK - 1 else (lane >= 0)
                iv = idx_v[pl.ds(o + ci * L, L)]
                wv = w_v[pl.ds(o + ci * L, L)]
                pv = iv >> 8
                paddr = (iv << 7) | bl
                pos = plsc.load_gather(cnt, [pv])
                dup, lastm = plsc.scan_count(pv, mask=m)
                slot = pos + dup - dbase
                plsc.store_scatter(af, [slot], paddr, mask=m)
                plsc.store_scatter(wf, [slot], wv, mask=m)
                plsc.store_scatter(cnt, [pv], slot + 1, mask=lastm & m)
            return c
        lax.fori_loop(0, RPW, place_row, 0)

        pltpu.sync_copy(af, a_hbm.at[pl.ds(w * FSTR, FSTR)])
        pltpu.sync_copy(wf, wf_hbm.at[pl.ds(w * FSTR, FSTR)])
        pltpu.sync_copy(off, p_hbm.at[pl.ds(w * PTAB, PTAB)])

    return ka


def _build_phase_b():
    info = plsc.get_sparse_core_info()
    nc = info.num_cores
    mesh = plsc.VectorSubcoreMesh(core_axis_name="c", subcore_axis_name="s")

    @functools.partial(
        pl.kernel,
        mesh=mesh,
        compiler_params=pltpu.CompilerParams(needs_layout_passes=False),
        out_type=jax.ShapeDtypeStruct((V, B), jnp.float32),
        scratch_types=[
            pltpu.VMEM((PIECE_V, 128), jnp.float32),
            pltpu.VMEM((PIECE_V, 128), jnp.float32),
            pltpu.VMEM((2 * 4 * SCAP,), jnp.int32),
            pltpu.VMEM((2 * 4 * SCAP,), jnp.float32),
            pltpu.VMEM((4 * PTAB,), jnp.int32),
            pltpu.SemaphoreType.DMA,
            pltpu.SemaphoreType.DMA,
            pltpu.SemaphoreType.DMA,
            pltpu.SemaphoreType.DMA,
        ],
    )
    def kb(a_hbm, wf_hbm, p_hbm, out_hbm, acc0, acc1, ast, wst, ptab,
           sg0, sg1, so0, so1):
        u = lax.axis_index("s") * nc + lax.axis_index("c")
        j = u & 7
        i = u >> 3
        lane = lax.broadcasted_iota(jnp.int32, (L,), 0)
        zf = jnp.zeros((L,), jnp.float32)

        for s in range(4):
            pltpu.sync_copy(p_hbm.at[pl.ds((4 * j + s) * PTAB, PTAB)],
                            ptab.at[pl.ds(s * PTAB, PTAB)])

        accs = (acc0, acc1)
        sgs = (sg0, sg1)
        sos = (so0, so1)

        def clear_full(acc):
            def zb(k, c):
                r = k >> 3
                col = (k & 7) * L
                acc[r, pl.ds(col, L)] = zf
                return c
            lax.fori_loop(0, (PIECE_V * 128) // L, zb, 0)

        clear_full(acc0)
        clear_full(acc1)

        def ext(s, p):
            ch = ptab[pl.ds(s * PTAB + (p & (-8)), L)]
            k = p & 7
            st = jnp.sum(jnp.where(lane == k, ch, 0))
            en = jnp.sum(jnp.where(lane == k + 1, ch, 0))
            return st, en - st

        def seg_params(s, p):
            st, ln = ext(s, p)
            return pl.multiple_of(st, 8), ln

        def gather_fire(q, p):
            sg = sgs[q]
            for s in range(4):
                st, ln = seg_params(s, p)
                fb = (4 * j + s) * FSTR

                @pl.when(ln > 0)
                def _():
                    pltpu.async_copy(
                        a_hbm.at[pl.ds(fb + st, SCAP)],
                        ast.at[pl.ds((q * 4 + s) * SCAP, SCAP)], sg)
                    pltpu.async_copy(
                        wf_hbm.at[pl.ds(fb + st, SCAP)],
                        wst.at[pl.ds((q * 4 + s) * SCAP, SCAP)], sg)

        def gather_drain(q, p):
            sg = sgs[q]
            for s in range(4):
                st, ln = seg_params(s, p)
                fb = (4 * j + s) * FSTR

                @pl.when(ln > 0)
                def _():
                    pltpu.make_async_copy(
                        a_hbm.at[pl.ds(fb + st, SCAP)],
                        ast.at[pl.ds((q * 4 + s) * SCAP, SCAP)], sg).wait()
                    pltpu.make_async_copy(
                        wf_hbm.at[pl.ds(fb + st, SCAP)],
                        wst.at[pl.ds((q * 4 + s) * SCAP, SCAP)], sg).wait()

        def scan_slot(q, s, p, nch, coff, add):
            acc = accs[q]
            base = (q * 4 + s) * SCAP + coff

            def cb(c, cc):
                av = ast[pl.ds(base + c * L, L)]
                m = (av >> 15) == p
                lv = (av >> 7) & (PIECE_V - 1)
                lb = av & 127
                if add:
                    wv = wst[pl.ds(base + c * L, L)]
                    plsc.addupdate_scatter(acc, [lv, lb], wv, mask=m)
                else:
                    plsc.store_scatter(acc, [lv, lb], zf, mask=m)
                return cc
            lax.fori_loop(0, nch, cb, 0)

        def scatter_piece(q, p):
            for s in range(4):
                st, ln = seg_params(s, p)
                nch0 = jnp.minimum(SCAP // L, (ln + L - 1) >> 4)
                scan_slot(q, s, p, nch0, 0, True)
                nb = (ln + SCAP - 1) >> SCAP_SHIFT
                fb = (4 * j + s) * FSTR

                def batch(k, c):
                    pltpu.sync_copy(
                        a_hbm.at[pl.ds(fb + st + k * SCAP, SCAP)],
                        ast.at[pl.ds((q * 4 + s) * SCAP, SCAP)])
                    pltpu.sync_copy(
                        wf_hbm.at[pl.ds(fb + st + k * SCAP, SCAP)],
                        wst.at[pl.ds((q * 4 + s) * SCAP, SCAP)])
                    rem = ln - k * SCAP
                    nchk = jnp.minimum(SCAP // L, (rem + L - 1) >> 4)
                    scan_slot(q, s, p, nchk, 0, True)
                    return c
                lax.fori_loop(1, nb, batch, 0)

        def rezero(q, p_prev):
            ovf = jnp.int32(0)
            lens = []
            for s in range(4):
                st, ln = seg_params(s, p_prev)
                lens.append(ln)
                ovf = ovf | (ln > SCAP).astype(jnp.int32)

            @pl.when(ovf == 1)
            def _():
                clear_full(accs[q])

            @pl.when(ovf == 0)
            def _():
                for s in range(4):
                    nch0 = jnp.minimum(SCAP // L, (lens[s] + L - 1) >> 4)
                    scan_slot(q, s, p_prev, nch0, 0, False)

        def out_fire(q, p, nrows):
            v0 = p * PIECE_V
            pltpu.async_copy(
                accs[q].at[pl.ds(0, nrows), :],
                out_hbm.at[pl.ds(v0, nrows), pl.ds(j * 128, 128)], sos[q])

        def out_drain(q, nrows):
            pltpu.make_async_copy(
                accs[q].at[pl.ds(0, nrows), :],
                out_hbm.at[pl.ds(0, nrows), pl.ds(j * 128, 128)],
                sos[q]).wait()

        def piece_of(t):
            return i + 4 * t

        def step(q, t, first):
            p = piece_of(t)
            if not first:
                out_drain(q, PIECE_V)
                rezero(q, p - 8)
            gather_fire(q, p)
            gather_drain(q, p)
            scatter_piece(q, p)
            out_fire(q, p, PIECE_V)

        step(0, jnp.int32(0), True)
        step(1, jnp.int32(1), True)

        def pair(t2, c):
            t = 2 + t2 * 2
            step(0, t, False)
            step(1, t + 1, False)
            return c
        lax.fori_loop(0, (NFULL - 3) // 2, pair, 0)
        step(0, jnp.int32(NFULL - 1), False)

        t97 = jnp.int32(NFULL)

        @pl.when(i <= 1)
        def _():
            step(1, t97, False)

        @pl.when(i == 2)
        def _():
            p = piece_of(t97)
            out_drain(1, PIECE_V)
            rezero(1, p - 8)
            gather_fire(1, p)
            gather_drain(1, p)
            scatter_piece(1, p)
            out_fire(1, p, VLAST)

        @pl.when(i <= 1)
        def _():
            out_drain(0, PIECE_V)
            out_drain(1, PIECE_V)

        @pl.when(i == 2)
        def _():
            out_drain(0, PIECE_V)
            out_drain(1, VLAST)

        @pl.when(i == 3)
        def _():
            out_drain(0, PIECE_V)
            out_drain(1, PIECE_V)

    return kb


_phase_a = _build_phase_a()
_phase_b = _build_phase_b()


@jax.jit
def kernel(inputs, count_weights):
    a_file, w_file, p_file = _phase_a(
        inputs.reshape(-1), count_weights.reshape(-1))
    out_t = _phase_b(a_file, w_file, p_file)
    return out_t.T
